# baseline (speedup 1.0000x reference)
import functools

import jax
import jax.numpy as jnp
from jax import lax
from jax.experimental import pallas as pl
from jax.experimental.pallas import tpu as pltpu
from jax.experimental.pallas import tpu_sc as plsc

NUM_POS_FEATS = 256
HALF = NUM_POS_FEATS // 2
LANES = 16


def _make_sc_lookup(b, h, w):
    info = plsc.get_sparse_core_info()
    nc, ns = info.num_cores, info.num_subcores
    nw = nc * ns
    assert h % nw == 0
    rpw = h // nw
    mesh = plsc.VectorSubcoreMesh(core_axis_name="c", subcore_axis_name="s")

    @functools.partial(
        pl.kernel,
        mesh=mesh,
        out_type=jax.ShapeDtypeStruct((b, h, w, NUM_POS_FEATS), jnp.float32),
        scratch_types=[
            pltpu.VMEM((w, HALF), jnp.float32),
            pltpu.VMEM((h, HALF), jnp.float32),
            pltpu.VMEM((rpw, w, NUM_POS_FEATS), jnp.float32),
            pltpu.SemaphoreType.DMA,
        ],
    )
    def sc_lookup(col_hbm, row_hbm, out_hbm, col_v, row_v, chunk, sem):
        wid = lax.axis_index("s") * nc + lax.axis_index("c")
        i0 = wid * rpw
        pltpu.sync_copy(col_hbm.at[pl.ds(0, w)], col_v)
        pltpu.sync_copy(row_hbm.at[pl.ds(0, h)], row_v)
        rv = [
            [row_v[i0 + r, pl.ds(LANES * k, LANES)] for k in range(HALF // LANES)]
            for r in range(rpw)
        ]

        def body(j, carry):
            for k in range(HALF // LANES):
                cv = col_v[j, pl.ds(LANES * k, LANES)]
                for r in range(rpw):
                    chunk[r, j, pl.ds(LANES * k, LANES)] = cv
            for r in range(rpw):
                for k in range(HALF // LANES):
                    chunk[r, j, pl.ds(HALF + LANES * k, LANES)] = rv[r][k]
            return carry

        lax.fori_loop(0, w, body, 0)
        pltpu.async_copy(chunk, out_hbm.at[0, pl.ds(i0, rpw)], sem).wait()

    return sc_lookup


def _make_tc_broadcast(b, h, w):
    def tc_broadcast(in_ref, out_ref, scratch, sem, sem2):
        pltpu.make_async_copy(in_ref.at[0], scratch, sem).start()
        pltpu.make_async_copy(in_ref.at[0], scratch, sem).wait()
        copies = [
            pltpu.make_async_copy(scratch, out_ref.at[i], sem2) for i in range(1, b)
        ]
        for c in copies:
            c.start()
        for c in copies:
            c.wait()

    return pl.pallas_call(
        tc_broadcast,
        in_specs=[pl.BlockSpec(memory_space=pl.ANY)],
        out_specs=pl.BlockSpec(memory_space=pl.ANY),
        out_shape=jax.ShapeDtypeStruct((b, h, w, NUM_POS_FEATS), jnp.float32),
        scratch_shapes=[
            pltpu.VMEM((h, w, NUM_POS_FEATS), jnp.float32),
            pltpu.SemaphoreType.DMA,
            pltpu.SemaphoreType.DMA,
        ],
        input_output_aliases={0: 0},
    )


def kernel(tensor_list, row_embed, col_embed):
    b, h, w = tensor_list.shape[0], tensor_list.shape[-3], tensor_list.shape[-2]
    seeded = _make_sc_lookup(b, h, w)(col_embed, row_embed)
    return _make_tc_broadcast(b, h, w)(seeded)


# speedup vs baseline: 1.3260x; 1.3260x over previous
import jax
import jax.numpy as jnp
from jax.experimental import pallas as pl
from jax.experimental.pallas import tpu as pltpu

NUM_POS_FEATS = 256


def _make_body(b, h, w):
    half = NUM_POS_FEATS // 2

    def _body(col_ref, row_ref, out_ref, scratch_a, scratch_b, sem_a, sem_b):
        col = col_ref[:w, :]
        row = row_ref[:h, :]
        xpart = jnp.broadcast_to(col[None, :, :], (h, w, half))
        ypart = jnp.broadcast_to(row[:, None, :], (h, w, half))
        scratch_a[:, :, :half] = xpart
        scratch_a[:, :, half:] = ypart
        scratch_b[:, :, :half] = xpart
        scratch_b[:, :, half:] = ypart
        copies = [
            pltpu.make_async_copy(
                scratch_a if i % 2 == 0 else scratch_b,
                out_ref.at[i],
                sem_a if i % 2 == 0 else sem_b,
            )
            for i in range(b)
        ]
        for c in copies:
            c.start()
        for c in copies:
            c.wait()

    return _body


def kernel(tensor_list, row_embed, col_embed):
    b, h, w = tensor_list.shape[0], tensor_list.shape[-3], tensor_list.shape[-2]
    out = pl.pallas_call(
        _make_body(b, h, w),
        in_specs=[
            pl.BlockSpec(memory_space=pltpu.VMEM),
            pl.BlockSpec(memory_space=pltpu.VMEM),
        ],
        out_specs=pl.BlockSpec(memory_space=pl.ANY),
        out_shape=jax.ShapeDtypeStruct((b, h, w, NUM_POS_FEATS), jnp.float32),
        scratch_shapes=[
            pltpu.VMEM((h, w, NUM_POS_FEATS), jnp.float32),
            pltpu.VMEM((h, w, NUM_POS_FEATS), jnp.float32),
            pltpu.SemaphoreType.DMA,
            pltpu.SemaphoreType.DMA,
        ],
    )(col_embed, row_embed)
    return out


# speedup vs baseline: 1.3417x; 1.0118x over previous
import jax
import jax.numpy as jnp
from jax.experimental import pallas as pl
from jax.experimental.pallas import tpu as pltpu

NUM_POS_FEATS = 256


def _make_body(b, h, w):
    half = NUM_POS_FEATS // 2

    def _body(col_ref, row_ref, out_ref, scratch, sem):
        col = col_ref[:w, :]
        row = row_ref[:h, :]
        scratch[:, :, :half] = jnp.broadcast_to(col[None, :, :], (h, w, half))
        scratch[:, :, half:] = jnp.broadcast_to(row[:, None, :], (h, w, half))
        copies = [
            pltpu.make_async_copy(scratch, out_ref.at[i], sem) for i in range(b)
        ]
        for c in copies:
            c.start()
        for c in copies:
            c.wait()

    return _body


def kernel(tensor_list, row_embed, col_embed):
    b, h, w = tensor_list.shape[0], tensor_list.shape[-3], tensor_list.shape[-2]
    out = pl.pallas_call(
        _make_body(b, h, w),
        in_specs=[
            pl.BlockSpec(memory_space=pltpu.VMEM),
            pl.BlockSpec(memory_space=pltpu.VMEM),
        ],
        out_specs=pl.BlockSpec(memory_space=pl.ANY),
        out_shape=jax.ShapeDtypeStruct((b, h, w, NUM_POS_FEATS), jnp.float32),
        scratch_shapes=[
            pltpu.VMEM((h, w, NUM_POS_FEATS), jnp.float32),
            pltpu.SemaphoreType.DMA,
        ],
    )(col_embed, row_embed)
    return out
